# Initial kernel scaffold; baseline (speedup 1.0000x reference)
import jax
import jax.numpy as jnp
from jax.experimental import pallas as pl


def kernel(x, A, W1, b1, W2, b2):
    raise NotImplementedError("write your pallas kernel here")


# speedup vs baseline: 8.0948x; 8.0948x over previous
import jax
import jax.numpy as jnp
from jax.experimental import pallas as pl


def _gcn2_body(x_ref, a_ref, w1_ref, b1_ref, w2_ref, b2_ref, o_ref):
    A = a_ref[0]
    x = x_ref[0]
    n = A.shape[0]

    rows = jax.lax.broadcasted_iota(jnp.int32, (n, n), 0)
    cols = jax.lax.broadcasted_iota(jnp.int32, (n, n), 1)
    eye = (rows == cols).astype(jnp.float32)
    diag = jnp.sum(A * eye, axis=0)
    mask = (diag == 0.0).astype(jnp.float32)
    deg = jnp.sum(A, axis=0) + mask
    dinv = jnp.where(deg > 0.0, jax.lax.rsqrt(deg), 0.0)
    dcol = dinv[:, None]

    xw = jnp.dot(x, w1_ref[...], preferred_element_type=jnp.float32)
    y = dcol * xw
    t = jax.lax.dot_general(A, y, (((0,), (0,)), ((), ())),
                            preferred_element_type=jnp.float32)
    t = t + mask[:, None] * y
    h = jnp.maximum(dcol * t + b1_ref[0], 0.0)

    hw = jnp.dot(h, w2_ref[...], preferred_element_type=jnp.float32)
    y2 = dcol * hw
    t2 = jax.lax.dot_general(A, y2, (((0,), (0,)), ((), ())),
                             preferred_element_type=jnp.float32)
    t2 = t2 + mask[:, None] * y2
    o_ref[0] = dcol * t2 + b2_ref[0]


def kernel(x, A, W1, b1, W2, b2):
    Bb, n, in_c = x.shape
    hid = W1.shape[1]
    out_c = W2.shape[1]
    b1r = b1.reshape(1, hid)
    b2r = b2.reshape(1, out_c)
    return pl.pallas_call(
        _gcn2_body,
        grid=(Bb,),
        in_specs=[
            pl.BlockSpec((1, n, in_c), lambda i: (i, 0, 0)),
            pl.BlockSpec((1, n, n), lambda i: (i, 0, 0)),
            pl.BlockSpec((in_c, hid), lambda i: (0, 0)),
            pl.BlockSpec((1, hid), lambda i: (0, 0)),
            pl.BlockSpec((hid, out_c), lambda i: (0, 0)),
            pl.BlockSpec((1, out_c), lambda i: (0, 0)),
        ],
        out_specs=pl.BlockSpec((1, n, out_c), lambda i: (i, 0, 0)),
        out_shape=jax.ShapeDtypeStruct((Bb, n, out_c), jnp.float32),
    )(x, A, W1, b1r, W2, b2r)


# speedup vs baseline: 8.2884x; 1.0239x over previous
import jax
import jax.numpy as jnp
from jax.experimental import pallas as pl


_DBLK = 128


def _gcn2_body(x_ref, a_ref, w1_ref, b1_ref, w2_ref, b2_ref, o_ref):
    A = a_ref[0]
    x = x_ref[0]
    n = A.shape[0]

    eye = (jax.lax.broadcasted_iota(jnp.int32, (_DBLK, _DBLK), 0)
           == jax.lax.broadcasted_iota(jnp.int32, (_DBLK, _DBLK), 1)
           ).astype(jnp.float32)
    diag = jnp.concatenate([
        jnp.sum(A[k * _DBLK:(k + 1) * _DBLK, k * _DBLK:(k + 1) * _DBLK] * eye,
                axis=0)
        for k in range(n // _DBLK)
    ])
    mask = (diag == 0.0).astype(jnp.float32)
    deg = jnp.sum(A, axis=0) + mask
    dinv = jnp.where(deg > 0.0, jax.lax.rsqrt(deg), 0.0)
    dcol = dinv[:, None]
    md = mask[:, None] * dcol

    As = (A * dcol).astype(jnp.bfloat16)

    xw = jnp.dot(x, w1_ref[...], preferred_element_type=jnp.float32)
    t = jax.lax.dot_general(As, xw.astype(jnp.bfloat16), (((0,), (0,)), ((), ())),
                            preferred_element_type=jnp.float32)
    t = t + md * xw
    h = jnp.maximum(dcol * t + b1_ref[0], 0.0)

    hw = jnp.dot(h, w2_ref[...], preferred_element_type=jnp.float32)
    t2 = jax.lax.dot_general(As, hw.astype(jnp.bfloat16), (((0,), (0,)), ((), ())),
                             preferred_element_type=jnp.float32)
    t2 = t2 + md * hw
    o_ref[0] = dcol * t2 + b2_ref[0]


def kernel(x, A, W1, b1, W2, b2):
    Bb, n, in_c = x.shape
    hid = W1.shape[1]
    out_c = W2.shape[1]
    b1r = b1.reshape(1, hid)
    b2r = b2.reshape(1, out_c)
    return pl.pallas_call(
        _gcn2_body,
        grid=(Bb,),
        in_specs=[
            pl.BlockSpec((1, n, in_c), lambda i: (i, 0, 0)),
            pl.BlockSpec((1, n, n), lambda i: (i, 0, 0)),
            pl.BlockSpec((in_c, hid), lambda i: (0, 0)),
            pl.BlockSpec((1, hid), lambda i: (0, 0)),
            pl.BlockSpec((hid, out_c), lambda i: (0, 0)),
            pl.BlockSpec((1, out_c), lambda i: (0, 0)),
        ],
        out_specs=pl.BlockSpec((1, n, out_c), lambda i: (i, 0, 0)),
        out_shape=jax.ShapeDtypeStruct((Bb, n, out_c), jnp.float32),
    )(x, A, W1, b1r, W2, b2r)


# speedup vs baseline: 9.2965x; 1.1216x over previous
import jax
import jax.numpy as jnp
from jax.experimental import pallas as pl

_NSLICE = 4


def _gcn2_body(*refs):
    x_ref, w1_ref, b1_ref, w2_ref, b2_ref = refs[:5]
    a_refs = refs[5:5 + _NSLICE]
    o_ref = refs[5 + _NSLICE]

    x = x_ref[0]
    a_slices = [r[0] for r in a_refs]
    n = x.shape[0]
    w = n // _NSLICE

    eye = (jax.lax.broadcasted_iota(jnp.int32, (w, w), 0)
           == jax.lax.broadcasted_iota(jnp.int32, (w, w), 1)
           ).astype(jnp.float32)
    diag = jnp.concatenate([
        jnp.sum(a_slices[j][j * w:(j + 1) * w, :] * eye, axis=0)
        for j in range(_NSLICE)
    ])
    mask = (diag == 0.0).astype(jnp.float32)
    deg = jnp.concatenate([jnp.sum(a, axis=0) for a in a_slices]) + mask
    dinv = jnp.where(deg > 0.0, jax.lax.rsqrt(deg), 0.0)
    dcol = dinv[:, None]
    md = mask[:, None] * dcol

    asc = [(a * dcol).astype(jnp.bfloat16) for a in a_slices]

    xw = jnp.dot(x, w1_ref[...], preferred_element_type=jnp.float32)
    xwb = xw.astype(jnp.bfloat16)
    t = jnp.concatenate([
        jax.lax.dot_general(a, xwb, (((0,), (0,)), ((), ())),
                            preferred_element_type=jnp.float32)
        for a in asc
    ])
    t = t + md * xw
    h = jnp.maximum(dcol * t + b1_ref[0], 0.0)

    hw = jnp.dot(h, w2_ref[...], preferred_element_type=jnp.float32)
    hwb = hw.astype(jnp.bfloat16)
    t2 = jnp.concatenate([
        jax.lax.dot_general(a, hwb, (((0,), (0,)), ((), ())),
                            preferred_element_type=jnp.float32)
        for a in asc
    ])
    t2 = t2 + md * hw
    o_ref[0] = dcol * t2 + b2_ref[0]


def kernel(x, A, W1, b1, W2, b2):
    Bb, n, in_c = x.shape
    hid = W1.shape[1]
    out_c = W2.shape[1]
    w = n // _NSLICE
    b1r = b1.reshape(1, hid)
    b2r = b2.reshape(1, out_c)

    a_specs = [
        pl.BlockSpec((1, n, w), lambda i, j=j: (i, 0, j))
        for j in range(_NSLICE)
    ]
    return pl.pallas_call(
        _gcn2_body,
        grid=(Bb,),
        in_specs=[
            pl.BlockSpec((1, n, in_c), lambda i: (i, 0, 0)),
            pl.BlockSpec((in_c, hid), lambda i: (0, 0)),
            pl.BlockSpec((1, hid), lambda i: (0, 0)),
            pl.BlockSpec((hid, out_c), lambda i: (0, 0)),
            pl.BlockSpec((1, out_c), lambda i: (0, 0)),
        ] + a_specs,
        out_specs=pl.BlockSpec((1, n, out_c), lambda i: (i, 0, 0)),
        out_shape=jax.ShapeDtypeStruct((Bb, n, out_c), jnp.float32),
    )(x, W1, b1r, W2, b2r, *([A] * _NSLICE))


# speedup vs baseline: 9.4205x; 1.0133x over previous
import jax
import jax.numpy as jnp
from jax.experimental import pallas as pl

_NSLICE = 4


def _gcn2_body(*refs):
    x_ref, w1_ref, b1_ref, w2_ref, b2_ref = refs[:5]
    a_refs = refs[5:5 + _NSLICE]
    o_ref = refs[5 + _NSLICE]

    x = x_ref[0]
    a_slices = [r[0] for r in a_refs]
    n = x.shape[0]
    w = n // _NSLICE

    eye = (jax.lax.broadcasted_iota(jnp.int32, (w, w), 0)
           == jax.lax.broadcasted_iota(jnp.int32, (w, w), 1)
           ).astype(jnp.float32)
    diag = jnp.concatenate([
        jnp.sum(a_slices[j][:, j * w:(j + 1) * w] * eye, axis=0)
        for j in range(_NSLICE)
    ])
    mask = (diag == 0.0).astype(jnp.float32)
    deg = sum(jnp.sum(a, axis=0) for a in a_slices) + mask
    dinv = jnp.where(deg > 0.0, jax.lax.rsqrt(deg), 0.0)
    dcol = dinv[:, None]
    md = mask[:, None] * dcol

    asc = [(a_slices[j] * dcol[j * w:(j + 1) * w]).astype(jnp.bfloat16)
           for j in range(_NSLICE)]

    def ahT_dot(yb):
        return sum(
            jax.lax.dot_general(asc[j], yb[j * w:(j + 1) * w],
                                (((0,), (0,)), ((), ())),
                                preferred_element_type=jnp.float32)
            for j in range(_NSLICE)
        )

    xw = jnp.dot(x, w1_ref[...], preferred_element_type=jnp.float32)
    t = ahT_dot(xw.astype(jnp.bfloat16)) + md * xw
    h = jnp.maximum(dcol * t + b1_ref[0], 0.0)

    hw = jnp.dot(h, w2_ref[...], preferred_element_type=jnp.float32)
    t2 = ahT_dot(hw.astype(jnp.bfloat16)) + md * hw
    o_ref[0] = dcol * t2 + b2_ref[0]


def kernel(x, A, W1, b1, W2, b2):
    Bb, n, in_c = x.shape
    hid = W1.shape[1]
    out_c = W2.shape[1]
    w = n // _NSLICE
    b1r = b1.reshape(1, hid)
    b2r = b2.reshape(1, out_c)

    a_specs = [
        pl.BlockSpec((1, w, n), lambda i, j=j: (i, j, 0))
        for j in range(_NSLICE)
    ]
    return pl.pallas_call(
        _gcn2_body,
        grid=(Bb,),
        in_specs=[
            pl.BlockSpec((1, n, in_c), lambda i: (i, 0, 0)),
            pl.BlockSpec((in_c, hid), lambda i: (0, 0)),
            pl.BlockSpec((1, hid), lambda i: (0, 0)),
            pl.BlockSpec((hid, out_c), lambda i: (0, 0)),
            pl.BlockSpec((1, out_c), lambda i: (0, 0)),
        ] + a_specs,
        out_specs=pl.BlockSpec((1, n, out_c), lambda i: (i, 0, 0)),
        out_shape=jax.ShapeDtypeStruct((Bb, n, out_c), jnp.float32),
    )(x, W1, b1r, W2, b2r, *([A] * _NSLICE))
